# Initial kernel scaffold; baseline (speedup 1.0000x reference)
import jax
import jax.numpy as jnp
from jax.experimental import pallas as pl


def kernel(hidden_states, wq, wk, wv, wo, q_scale, k_scale, router_w, w_up, b_up, w_down, b_down):
    raise NotImplementedError("write your pallas kernel here")


# speedup vs baseline: 1.4665x; 1.4665x over previous
import functools

import jax
import jax.numpy as jnp
import numpy as np
from jax.experimental import pallas as pl
from jax.experimental.pallas import tpu as pltpu

B, S, H = 1, 2048, 2048
HQ, HKV, D = 16, 4, 128
E, I = 16, 768
EPS = 1e-6
ROPE_THETA = 10000.0
NEG = float(np.finfo(np.float32).min)

HP = None

BT_QKV = 256


def _qkv_kernel(x_ref, wq_ref, wk_ref, wv_ref, qs_ref, ks_ref, cos_ref,
                sin_ref, q_out, k_out, v_out):
    xb = x_ref[...]
    cos = cos_ref[...][:, None, :]
    sin = sin_ref[...][:, None, :]

    def norm_rope(y, nh, scale):
        y = y.reshape(BT_QKV, nh, D)
        var = jnp.mean(y * y, axis=-1, keepdims=True)
        y = y * jax.lax.rsqrt(var + EPS) * scale.reshape(1, 1, D)
        half = D // 2
        rot = jnp.concatenate([-y[..., half:], y[..., :half]], axis=-1)
        y = y * cos + rot * sin
        return y.reshape(BT_QKV, nh * D)

    q = jnp.dot(xb, wq_ref[...], preferred_element_type=jnp.float32,
                precision=HP)
    q_out[...] = norm_rope(q, HQ, qs_ref[...])
    k = jnp.dot(xb, wk_ref[...], preferred_element_type=jnp.float32,
                precision=HP)
    k_out[...] = norm_rope(k, HKV, ks_ref[...])
    v_out[...] = jnp.dot(xb, wv_ref[...], preferred_element_type=jnp.float32,
                         precision=HP)


def _qkv(x, wq, wk, wv, q_scale, k_scale, cos, sin):
    nb = S // BT_QKV
    return pl.pallas_call(
        _qkv_kernel,
        grid=(nb,),
        in_specs=[
            pl.BlockSpec((BT_QKV, H), lambda i: (i, 0)),
            pl.BlockSpec((H, HQ * D), lambda i: (0, 0)),
            pl.BlockSpec((H, HKV * D), lambda i: (0, 0)),
            pl.BlockSpec((H, HKV * D), lambda i: (0, 0)),
            pl.BlockSpec((1, D), lambda i: (0, 0)),
            pl.BlockSpec((1, D), lambda i: (0, 0)),
            pl.BlockSpec((BT_QKV, D), lambda i: (i, 0)),
            pl.BlockSpec((BT_QKV, D), lambda i: (i, 0)),
        ],
        out_specs=[
            pl.BlockSpec((BT_QKV, HQ * D), lambda i: (i, 0)),
            pl.BlockSpec((BT_QKV, HKV * D), lambda i: (i, 0)),
            pl.BlockSpec((BT_QKV, HKV * D), lambda i: (i, 0)),
        ],
        out_shape=[
            jax.ShapeDtypeStruct((S, HQ * D), jnp.float32),
            jax.ShapeDtypeStruct((S, HKV * D), jnp.float32),
            jax.ShapeDtypeStruct((S, HKV * D), jnp.float32),
        ],
        compiler_params=pltpu.CompilerParams(
            dimension_semantics=("parallel",)),
        name="qkv_rope",
    )(x, wq, wk, wv, q_scale, k_scale, cos, sin)


BQ = 512
BKV = 512


def _attn_kernel(q_ref, k_ref, v_ref, o_ref, s_scr, m_scr, l_scr, acc_scr):
    i = pl.program_id(1)
    q = q_ref[...]
    nkv = S // BKV
    inv_sqrt_d = 1.0 / np.sqrt(D)

    m_scr[...] = jnp.full_like(m_scr, NEG)
    for j in range(nkv):
        sl = slice(j * BKV, (j + 1) * BKV)

        @pl.when(j <= i)
        def _(j=j, sl=sl):
            kj = k_ref[sl, :]
            s = jax.lax.dot_general(q, kj, (((1,), (1,)), ((), ())),
                                    preferred_element_type=jnp.float32,
                                    precision=HP) * inv_sqrt_d
            qpos = i * BQ + jax.lax.broadcasted_iota(jnp.int32, (BQ, BKV), 0)
            kpos = j * BKV + jax.lax.broadcasted_iota(jnp.int32, (BQ, BKV), 1)
            s = jnp.where(qpos >= kpos, s, NEG)
            s_scr[:, sl] = s
            m_scr[...] = jnp.maximum(m_scr[...],
                                     jnp.max(s, axis=-1, keepdims=True))

    l_scr[...] = jnp.zeros_like(l_scr)
    for j in range(nkv):
        sl = slice(j * BKV, (j + 1) * BKV)

        @pl.when(j <= i)
        def _(j=j, sl=sl):
            p = jnp.exp(s_scr[:, sl] - m_scr[...][:, :1])
            s_scr[:, sl] = p
            l_scr[...] = l_scr[...] + jnp.sum(p, axis=-1, keepdims=True)

    acc_scr[...] = jnp.zeros_like(acc_scr)
    for j in range(nkv):
        sl = slice(j * BKV, (j + 1) * BKV)

        @pl.when(j <= i)
        def _(j=j, sl=sl):
            probs = s_scr[:, sl] / l_scr[...][:, :1]
            acc_scr[...] = acc_scr[...] + jax.lax.dot_general(
                probs, v_ref[sl, :], (((1,), (0,)), ((), ())),
                preferred_element_type=jnp.float32, precision=HP)
    o_ref[...] = acc_scr[...]


def _attention(q, k, v):
    nq = S // BQ
    return pl.pallas_call(
        _attn_kernel,
        grid=(HQ, nq),
        in_specs=[
            pl.BlockSpec((BQ, D), lambda h, i: (i, h)),
            pl.BlockSpec((S, D), lambda h, i: (0, h // (HQ // HKV))),
            pl.BlockSpec((S, D), lambda h, i: (0, h // (HQ // HKV))),
        ],
        out_specs=pl.BlockSpec((BQ, D), lambda h, i: (i, h)),
        out_shape=jax.ShapeDtypeStruct((S, HQ * D), jnp.float32),
        scratch_shapes=[
            pltpu.VMEM((BQ, S), jnp.float32),
            pltpu.VMEM((BQ, 128), jnp.float32),
            pltpu.VMEM((BQ, 128), jnp.float32),
            pltpu.VMEM((BQ, D), jnp.float32),
        ],
        compiler_params=pltpu.CompilerParams(
            dimension_semantics=("parallel", "parallel")),
        name="attn2pass",
    )(q, k, v)


BT_PR = 256


def _proj_router_kernel(a_ref, x_ref, wo_ref, rw_ref, h_out, lg_out,
                        mask_out):
    a = a_ref[...]
    hb = jnp.dot(a, wo_ref[...], preferred_element_type=jnp.float32,
                 precision=HP) + x_ref[...]
    h_out[...] = hb
    lg = jnp.dot(hb, rw_ref[...], preferred_element_type=jnp.float32,
                 precision=HP)
    lg_out[...] = lg
    idx = jax.lax.broadcasted_iota(jnp.int32, (BT_PR, E), 1)
    m1 = jnp.max(lg, axis=-1, keepdims=True)
    i1 = jnp.min(jnp.where(lg == m1, idx, E), axis=-1, keepdims=True)
    lg2 = jnp.where(idx == i1, NEG, lg)
    m2 = jnp.max(lg2, axis=-1, keepdims=True)
    i2 = jnp.min(jnp.where(lg2 == m2, idx, E), axis=-1, keepdims=True)
    mask_out[...] = ((idx == i1) | (idx == i2)).astype(jnp.float32)


def _proj_router(attn, x, wo, router_w):
    nb = S // BT_PR
    return pl.pallas_call(
        _proj_router_kernel,
        grid=(nb,),
        in_specs=[
            pl.BlockSpec((BT_PR, HQ * D), lambda i: (i, 0)),
            pl.BlockSpec((BT_PR, H), lambda i: (i, 0)),
            pl.BlockSpec((HQ * D, H), lambda i: (0, 0)),
            pl.BlockSpec((H, E), lambda i: (0, 0)),
        ],
        out_specs=[
            pl.BlockSpec((BT_PR, H), lambda i: (i, 0)),
            pl.BlockSpec((BT_PR, E), lambda i: (i, 0)),
            pl.BlockSpec((BT_PR, E), lambda i: (i, 0)),
        ],
        out_shape=[
            jax.ShapeDtypeStruct((S, H), jnp.float32),
            jax.ShapeDtypeStruct((S, E), jnp.float32),
            jax.ShapeDtypeStruct((S, E), jnp.float32),
        ],
        compiler_params=pltpu.CompilerParams(
            dimension_semantics=("parallel",)),
        name="proj_router",
    )(attn, x, wo, router_w)


TB_MOE = 512


def _moe_kernel(hf_ref, mask_ref, wu_ref, bu_ref, wd_ref, bd_ref, o_ref,
                hb_scr):
    e = pl.program_id(1)

    @pl.when(e == 0)
    def _():
        hb_scr[...] = hf_ref[...].astype(jnp.bfloat16)

    up = jnp.dot(hb_scr[...], wu_ref[0],
                 preferred_element_type=jnp.float32) + bu_ref[0]
    up = up * jax.nn.sigmoid(up)
    lane = jax.lax.broadcasted_iota(jnp.int32, (TB_MOE, E), 1)
    col = jnp.sum(jnp.where(lane == e, mask_ref[...], 0.0), axis=1,
                  keepdims=True)
    upm = (up * col).astype(jnp.bfloat16)
    dn = jnp.dot(upm, wd_ref[0], preferred_element_type=jnp.float32)

    @pl.when(e == 0)
    def _():
        o_ref[...] = (hf_ref[...] + dn
                      + jnp.dot(mask_ref[...], bd_ref[...],
                                preferred_element_type=jnp.float32))

    @pl.when(e > 0)
    def _():
        o_ref[...] = o_ref[...] + dn


def _moe(h, mask, w_up, b_up, w_down, b_down):
    nt = S // TB_MOE
    return pl.pallas_call(
        _moe_kernel,
        grid=(nt, E),
        in_specs=[
            pl.BlockSpec((TB_MOE, H), lambda t, e: (t, 0)),
            pl.BlockSpec((TB_MOE, E), lambda t, e: (t, 0)),
            pl.BlockSpec((1, H, I), lambda t, e: (e, 0, 0)),
            pl.BlockSpec((1, 1, I), lambda t, e: (e, 0, 0)),
            pl.BlockSpec((1, I, H), lambda t, e: (e, 0, 0)),
            pl.BlockSpec((E, H), lambda t, e: (0, 0)),
        ],
        out_specs=pl.BlockSpec((TB_MOE, H), lambda t, e: (t, 0)),
        out_shape=jax.ShapeDtypeStruct((S, H), jnp.float32),
        scratch_shapes=[pltpu.VMEM((TB_MOE, H), jnp.bfloat16)],
        compiler_params=pltpu.CompilerParams(
            dimension_semantics=("parallel", "arbitrary")),
        name="moe_ffn",
    )(h, mask, w_up, b_up, w_down, b_down)


def kernel(hidden_states, wq, wk, wv, wo, q_scale, k_scale, router_w, w_up,
           b_up, w_down, b_down):
    x = hidden_states.reshape(S, H)
    pos = jnp.arange(S)
    inv_freq = 1.0 / (ROPE_THETA ** (jnp.arange(0, D, 2, dtype=jnp.float32)
                                     / D))
    ang = pos[:, None].astype(jnp.float32) * inv_freq[None, :]
    cos = jnp.concatenate([jnp.cos(ang), jnp.cos(ang)], -1)
    sin = jnp.concatenate([jnp.sin(ang), jnp.sin(ang)], -1)

    q, k, v = _qkv(x, wq, wk, wv, q_scale.reshape(1, D),
                   k_scale.reshape(1, D), cos, sin)
    attn = _attention(q, k, v)
    h, router_logits, mask = _proj_router(attn, x, wo, router_w)
    out = _moe(h, mask, w_up.astype(jnp.bfloat16),
               b_up.reshape(E, 1, I), w_down.astype(jnp.bfloat16), b_down)
    return out.reshape(B, S, H), router_logits
